# baseline (speedup 1.0000x reference)
import functools

import jax
import jax.numpy as jnp
from jax import lax
from jax.experimental import pallas as pl
from jax.experimental.pallas import tpu as pltpu
from jax.experimental.pallas import tpu_sc as plsc

B, K = 16384, 100
NW = 32
COLS_W = B // NW
RC = 128
N_ROUNDS = COLS_W // RC
ST_CH = 5000
N_CH = 1000000 // ST_CH


@jax.jit
def _sc_gather(v, idx_t):
    mesh = plsc.VectorSubcoreMesh(core_axis_name="c", subcore_axis_name="s")

    @functools.partial(
        pl.kernel,
        mesh=mesh,
        out_type=jax.ShapeDtypeStruct((K, B), jnp.float32),
        scratch_types=[
            pltpu.VMEM_SHARED((1000000,), jnp.float32),
            pltpu.VMEM((K, RC), jnp.int32),
            pltpu.VMEM((K, RC), jnp.int32),
            pltpu.VMEM((K, RC), jnp.float32),
            pltpu.VMEM((K, RC), jnp.float32),
            pltpu.VMEM((ST_CH,), jnp.float32),
            pltpu.VMEM((ST_CH,), jnp.float32),
            pltpu.SemaphoreType.DMA,
            pltpu.SemaphoreType.DMA,
            pltpu.SemaphoreType.DMA,
            pltpu.SemaphoreType.DMA,
        ],
    )
    def k(v_hbm, idx_hbm, out_hbm, vs, idx_a, idx_b, out_a, out_b, tmp_a,
          tmp_b, sem_st, sem_i, sem_g, sem_o):
        sid = lax.axis_index("s")
        wid = sid * 2 + lax.axis_index("c")
        col0 = wid * COLS_W

        idx_bufs = [idx_a, idx_b]
        out_bufs = [out_a, out_b]

        def idx_slice(h):
            return idx_hbm.at[:, pl.ds(col0 + h * RC, RC)]

        def out_slice(h):
            return out_hbm.at[:, pl.ds(col0 + h * RC, RC)]

        pltpu.async_copy(idx_slice(0), idx_a, sem_i)

        with jax.named_scope("stage_v"):
            @pl.loop(sid, N_CH, step=32)
            def _stage(c):
                c2 = c + 16
                pltpu.async_copy(v_hbm.at[pl.ds(c * ST_CH, ST_CH)], tmp_a,
                                 sem_st)

                @pl.when(c2 < N_CH)
                def _():
                    pltpu.async_copy(v_hbm.at[pl.ds(c2 * ST_CH, ST_CH)],
                                     tmp_b, sem_st)

                pltpu.make_async_copy(v_hbm.at[pl.ds(c * ST_CH, ST_CH)],
                                      tmp_a, sem_st).wait()
                pltpu.sync_copy(tmp_a, vs.at[pl.ds(c * ST_CH, ST_CH)])

                @pl.when(c2 < N_CH)
                def _():
                    pltpu.make_async_copy(v_hbm.at[pl.ds(c2 * ST_CH, ST_CH)],
                                          tmp_b, sem_st).wait()
                    pltpu.sync_copy(tmp_b, vs.at[pl.ds(c2 * ST_CH, ST_CH)])

            plsc.subcore_barrier()

        gsems = [sem_g, sem_st]

        def fire(h):
            cur_idx, cur_out, gs = idx_bufs[h % 2], out_bufs[h % 2], gsems[h % 2]
            with jax.named_scope("fire"):

                @pl.loop(0, K, unroll=4)
                def _fire(r):
                    pltpu.async_copy(vs.at[cur_idx.at[r]], cur_out.at[r], gs)

        def drain(h):
            with jax.named_scope("drain"):
                pltpu.make_async_copy(out_slice(h), out_bufs[h % 2],
                                      gsems[h % 2]).wait()

        pltpu.make_async_copy(idx_slice(0), idx_a, sem_i).wait()
        pltpu.async_copy(idx_slice(1), idx_b, sem_i)
        fire(0)
        for h in range(1, N_ROUNDS):
            with jax.named_scope("idx_wait"):
                pltpu.make_async_copy(idx_slice(h), idx_bufs[h % 2],
                                      sem_i).wait()
            if h >= 2:
                with jax.named_scope("store_wait"):
                    pltpu.make_async_copy(out_bufs[h % 2], out_slice(h - 2),
                                          sem_o).wait()
            drain(h - 1)
            pltpu.async_copy(out_bufs[(h - 1) % 2], out_slice(h - 1), sem_o)
            if h + 1 < N_ROUNDS:
                pltpu.async_copy(idx_slice(h + 1), idx_bufs[(h + 1) % 2],
                                 sem_i)
            fire(h)

        drain(N_ROUNDS - 1)
        pltpu.async_copy(out_bufs[(N_ROUNDS - 1) % 2],
                         out_slice(N_ROUNDS - 1), sem_o)
        with jax.named_scope("tail_waits"):
            pltpu.make_async_copy(out_bufs[(N_ROUNDS - 2) % 2],
                                  out_slice(N_ROUNDS - 2), sem_o).wait()
            pltpu.make_async_copy(out_bufs[(N_ROUNDS - 1) % 2],
                                  out_slice(N_ROUNDS - 1), sem_o).wait()

    return k(v, idx_t)


def kernel(idx, v):
    out_t = _sc_gather(v, idx.astype(jnp.int32).T)
    return out_t.T


# speedup vs baseline: 1.1205x; 1.1205x over previous
import jax
import jax.numpy as jnp
from jax import lax
from jax.experimental import pallas as pl
from jax.experimental.pallas import tpu as pltpu
from jax.experimental.pallas import tpu_sc as plsc
from jax._src.pallas import mpmd

B, K = 16384, 100
NW = 32
COLS_W = B // NW
RC = 128
N_ROUNDS = COLS_W // RC
NV = 1000000


def _scs_fn(v_hbm, idx_hbm, out_hbm, vs, idx_a, idx_b, out_a, out_b,
            sem_i, sem_ga, sem_gb, sem_o, sem_scs, sem_rdy):
    del idx_hbm, out_hbm, idx_a, idx_b, out_a, out_b, sem_i, sem_ga
    del sem_gb, sem_o
    pltpu.async_copy(v_hbm, vs, sem_scs).wait()

    @pl.loop(0, 16)
    def _signal(i):
        pltpu.semaphore_signal(sem_rdy, 1, device_id={"s": i})


def _tec_fn(v_hbm, idx_hbm, out_hbm, vs, idx_a, idx_b, out_a, out_b,
            sem_i, sem_ga, sem_gb, sem_o, sem_scs, sem_rdy):
    del v_hbm, sem_scs
    sid = lax.axis_index("s")
    wid = sid * 2 + lax.axis_index("c")
    col0 = wid * COLS_W

    idx_bufs = [idx_a, idx_b]
    out_bufs = [out_a, out_b]
    gsems = [sem_ga, sem_gb]

    def idx_slice(h):
        return idx_hbm.at[:, pl.ds(col0 + h * RC, RC)]

    def out_slice(h):
        return out_hbm.at[:, pl.ds(col0 + h * RC, RC)]

    def fire(h):
        cur_idx, cur_out, gs = idx_bufs[h % 2], out_bufs[h % 2], gsems[h % 2]
        with jax.named_scope("fire"):

            @pl.loop(0, K, unroll=4)
            def _fire(r):
                pltpu.async_copy(vs.at[cur_idx.at[r]], cur_out.at[r], gs)

    def drain(h):
        with jax.named_scope("drain"):
            pltpu.make_async_copy(out_slice(h), out_bufs[h % 2],
                                  gsems[h % 2]).wait()

    pltpu.async_copy(idx_slice(0), idx_a, sem_i)
    pltpu.async_copy(idx_slice(1), idx_b, sem_i)
    pltpu.make_async_copy(idx_slice(0), idx_a, sem_i).wait()
    with jax.named_scope("stage_wait"):
        pltpu.semaphore_wait(sem_rdy, 1)

    fire(0)
    for h in range(1, N_ROUNDS):
        with jax.named_scope("idx_wait"):
            pltpu.make_async_copy(idx_slice(h), idx_bufs[h % 2], sem_i).wait()
        if h >= 2:
            with jax.named_scope("store_wait"):
                pltpu.make_async_copy(out_bufs[h % 2], out_slice(h - 2),
                                      sem_o).wait()
        drain(h - 1)
        pltpu.async_copy(out_bufs[(h - 1) % 2], out_slice(h - 1), sem_o)
        if h + 1 < N_ROUNDS:
            pltpu.async_copy(idx_slice(h + 1), idx_bufs[(h + 1) % 2], sem_i)
        fire(h)

    drain(N_ROUNDS - 1)
    pltpu.async_copy(out_bufs[(N_ROUNDS - 1) % 2], out_slice(N_ROUNDS - 1),
                     sem_o)
    with jax.named_scope("tail_waits"):
        pltpu.make_async_copy(out_bufs[(N_ROUNDS - 2) % 2],
                              out_slice(N_ROUNDS - 2), sem_o).wait()
        pltpu.make_async_copy(out_bufs[(N_ROUNDS - 1) % 2],
                              out_slice(N_ROUNDS - 1), sem_o).wait()


@jax.jit
def _sc_gather(v, idx_t):
    scalar_mesh = plsc.ScalarSubcoreMesh(axis_name="c", num_cores=2)
    vector_mesh = plsc.VectorSubcoreMesh(core_axis_name="c",
                                         subcore_axis_name="s")
    f = mpmd.mpmd_map(
        [(scalar_mesh, _scs_fn), (vector_mesh, _tec_fn)],
        out_types=jax.ShapeDtypeStruct((K, B), jnp.float32),
        scratch_types=[
            pltpu.VMEM_SHARED((NV,), jnp.float32),
            pltpu.VMEM((K, RC), jnp.int32) @ vector_mesh,
            pltpu.VMEM((K, RC), jnp.int32) @ vector_mesh,
            pltpu.VMEM((K, RC), jnp.float32) @ vector_mesh,
            pltpu.VMEM((K, RC), jnp.float32) @ vector_mesh,
            pltpu.SemaphoreType.DMA @ vector_mesh,
            pltpu.SemaphoreType.DMA @ vector_mesh,
            pltpu.SemaphoreType.DMA @ vector_mesh,
            pltpu.SemaphoreType.DMA @ vector_mesh,
            pltpu.SemaphoreType.DMA @ scalar_mesh,
            pltpu.SemaphoreType.REGULAR @ vector_mesh,
        ],
    )
    return f(v, idx_t)


def kernel(idx, v):
    out_t = _sc_gather(v, idx.astype(jnp.int32).T)
    return out_t.T
